# Initial kernel scaffold; baseline (speedup 1.0000x reference)
import jax
import jax.numpy as jnp
from jax.experimental import pallas as pl


def kernel(coords, values, w):
    raise NotImplementedError("write your pallas kernel here")


# speedup vs baseline: 4.6231x; 4.6231x over previous
import functools

import jax
import jax.numpy as jnp
from jax.experimental import pallas as pl
from jax.experimental.pallas import tpu as pltpu

B, S, D = 2, 3, 16
WMINOR = 384
BASE = 32
ACC_LO, ACC_HI = 32, 352


def _conv_body(w_ref, grid_ref, mask_ref, out_ref):
    i = pl.program_id(1)
    j = pl.program_id(2)

    @pl.when((i == 0) & (j == 0))
    def _init():
        out_ref[...] = jnp.zeros_like(out_ref)

    def tap_block(k, acc):
        t1 = k // 9
        t2 = (k // 3) % 3
        t3 = k % 3
        sub = grid_ref[0, i + t1, j + t2, pl.ds(t3, D), :, :]
        tbase = k * 27
        for t4 in range(3):
            for t5 in range(3):
                for t6 in range(3):
                    dlt = (t5 - 1) * 18 + (t6 - 1)
                    t = tbase + t4 * 9 + t5 * 3 + t6
                    wv = jnp.stack([w_ref[0, t], w_ref[1, t]])
                    src = sub[:, t4:t4 + D, ACC_LO + dlt:ACC_HI + dlt]
                    acc = acc + wv[:, None, None, None] * src[None]
        return acc

    acc = jax.lax.fori_loop(
        0, 27, tap_block,
        jnp.zeros((2, D, D, ACC_HI - ACC_LO), jnp.float32))
    m = mask_ref[0, 0, 0, :, :, ACC_LO:ACC_HI]
    s = (jax.nn.sigmoid(acc[0]) + jax.nn.sigmoid(acc[1])) * m
    out_ref[0, :, :, ACC_LO:ACC_HI] += s


@functools.partial(jax.jit, static_argnames=())
def kernel(coords, values, w):
    feats = jax.nn.relu(values)
    b = coords[:, 0]
    col = BASE + (coords[:, 5] + 1) * 18 + (coords[:, 6] + 1)
    grid = jnp.zeros((B, S + 2, S + 2, D + 2, D + 2, WMINOR), jnp.float32)
    grid = grid.at[b, coords[:, 1] + 1, coords[:, 2] + 1,
                   coords[:, 3] + 1, coords[:, 4] + 1, col].set(feats)
    mask = jnp.zeros((B, S, S, D, D, WMINOR), jnp.float32)
    mask = mask.at[b, coords[:, 1], coords[:, 2],
                   coords[:, 3], coords[:, 4], col].set(1.0)

    w2 = jnp.transpose(w, (1, 0, 4, 5, 2, 3))
    wpair = jnp.stack([w.reshape(-1), w2.reshape(-1)])

    out = pl.pallas_call(
        _conv_body,
        grid=(B, S, S),
        in_specs=[
            pl.BlockSpec((2, 729), lambda bb, ii, jj: (0, 0),
                         memory_space=pltpu.SMEM),
            pl.BlockSpec((1, S + 2, S + 2, D + 2, D + 2, WMINOR),
                         lambda bb, ii, jj: (bb, 0, 0, 0, 0, 0)),
            pl.BlockSpec((1, 1, 1, D, D, WMINOR),
                         lambda bb, ii, jj: (bb, ii, jj, 0, 0, 0)),
        ],
        out_specs=pl.BlockSpec((1, D, D, WMINOR),
                               lambda bb, ii, jj: (bb, 0, 0, 0)),
        out_shape=jax.ShapeDtypeStruct((B, D, D, WMINOR), jnp.float32),
    )(wpair, grid, mask)

    return out[..., 51:339].reshape(B, D, D, D, 18)[..., :D]
